# Initial kernel scaffold; baseline (speedup 1.0000x reference)
import functools

import jax
import jax.numpy as jnp
from jax import lax
from jax.experimental import pallas as pl
from jax.experimental.pallas import tpu as pltpu
from jax.experimental.pallas import tpu_sc as plsc

N_SRC = 10000
N_DST = 10000
E = 320000
D_FEAT = 128
H = 8
D = 16
HD = H * D
NEG_SLOPE = 0.2

NC = 2
NS = 16
NW = NC * NS
EPW = E // NW
C = 80
NCH = EPW // C
STRIPE = N_DST // NS


def _proj_body(fsrc, fdst, Ws, bs, Wd, bd, asrc, adst, sel,
               fs_out, el_out, er_out):
    fs = jnp.dot(fsrc[...], Ws[...], preferred_element_type=jnp.float32)
    fs = fs + bs[...]
    fd = jnp.dot(fdst[...], Wd[...], preferred_element_type=jnp.float32)
    fd = fd + bd[...]
    fs_out[...] = fs
    el_out[...] = jnp.dot(fs * asrc[...], sel[...],
                          preferred_element_type=jnp.float32)
    er_out[...] = jnp.dot(fd * adst[...], sel[...],
                          preferred_element_type=jnp.float32)


def _bcast_lane(v, h):
    return jnp.take(v, jnp.full((16,), h, dtype=jnp.int32),
                    mode="promise_in_bounds")


def _edge_body(fs_hbm, el_hbm, er_hbm, src_hbm, dst_hbm,
               o_hbm, d_hbm,
               out_sh, den_sh, src_all, dst_all, elg, erg, rows, sem):
    c = lax.axis_index("c")
    s = lax.axis_index("s")
    wid = c * NS + s

    def _zrows(i, _):
        for h in range(8):
            rows[i, pl.ds(16 * h, 16)] = jnp.zeros((16,), jnp.float32)
        return 0
    lax.fori_loop(0, C, _zrows, 0)

    def _zelg(i, _):
        elg[i, :] = jnp.zeros((16,), jnp.float32)
        return 0
    lax.fori_loop(0, C, _zelg, 0)

    def _zo(t, _):
        pltpu.sync_copy(rows, out_sh.at[pl.ds(s * STRIPE + t * C, C)])
        return 0
    lax.fori_loop(0, 7, _zo, 0)
    pltpu.sync_copy(rows.at[pl.ds(0, STRIPE - 7 * C)],
                    out_sh.at[pl.ds(s * STRIPE + 7 * C, STRIPE - 7 * C)])

    def _zd(t, _):
        pltpu.sync_copy(elg, den_sh.at[pl.ds(s * STRIPE + t * C, C)])
        return 0
    lax.fori_loop(0, 7, _zd, 0)
    pltpu.sync_copy(elg.at[pl.ds(0, STRIPE - 7 * C)],
                    den_sh.at[pl.ds(s * STRIPE + 7 * C, STRIPE - 7 * C)])

    pltpu.sync_copy(src_hbm.at[wid], src_all)
    pltpu.sync_copy(dst_hbm.at[wid], dst_all)

    plsc.subcore_barrier()

    def _chunk(j, _):
        src_i = src_all.at[j]
        dst_i = dst_all.at[j]
        pltpu.async_copy(el_hbm.at[src_i], elg, sem).wait()
        pltpu.async_copy(er_hbm.at[dst_i], erg, sem).wait()
        pltpu.async_copy(fs_hbm.at[src_i], rows, sem).wait()

        def _wl(i, _):
            e = elg[i, :] + erg[i, :]
            e = jnp.maximum(e, 0.0) + NEG_SLOPE * jnp.minimum(e, 0.0)
            elg[i, :] = jnp.exp(e)
            return 0
        lax.fori_loop(0, C, _wl, 0)

        def _ml(i, _):
            wv = elg[i, :]
            for h in range(8):
                seg = rows[i, pl.ds(16 * h, 16)]
                rows[i, pl.ds(16 * h, 16)] = seg * _bcast_lane(wv, h)
            return 0
        lax.fori_loop(0, C, _ml, 0)

        pltpu.sync_copy(rows, out_sh.at[dst_i], add=True)
        pltpu.sync_copy(elg, den_sh.at[dst_i], add=True)
        return 0
    lax.fori_loop(0, NCH, _chunk, 0)

    plsc.subcore_barrier()

    pltpu.sync_copy(out_sh.at[pl.ds(s * STRIPE, STRIPE)],
                    o_hbm.at[c, pl.ds(s * STRIPE, STRIPE)])
    pltpu.sync_copy(den_sh.at[pl.ds(s * STRIPE, STRIPE)],
                    d_hbm.at[c, pl.ds(s * STRIPE, STRIPE)])


_edge_pass = functools.partial(
    pl.kernel,
    out_type=(
        jax.ShapeDtypeStruct((NC, N_DST, HD), jnp.float32),
        jax.ShapeDtypeStruct((NC, N_DST, 16), jnp.float32),
    ),
    mesh=plsc.VectorSubcoreMesh(core_axis_name="c", subcore_axis_name="s"),
    scratch_types=[
        pltpu.VMEM_SHARED((N_DST, HD), jnp.float32),
        pltpu.VMEM_SHARED((N_DST, 16), jnp.float32),
        pltpu.VMEM((NCH, C), jnp.int32),
        pltpu.VMEM((NCH, C), jnp.int32),
        pltpu.VMEM((C, 16), jnp.float32),
        pltpu.VMEM((C, 16), jnp.float32),
        pltpu.VMEM((C, HD), jnp.float32),
        pltpu.SemaphoreType.DMA,
    ],
)(_edge_body)


def _final_body(o_ref, d_ref, selT, out_ref):
    o = o_ref[0] + o_ref[1]
    den = d_ref[0] + d_ref[1]
    den128 = jnp.dot(den, selT[...], preferred_element_type=jnp.float32)
    out_ref[...] = o / (den128 + 1e-16)


def kernel(feat_src, feat_dst, edge_index, W_src, b_src, W_dst, b_dst,
           attn_src):
    f32 = jnp.float32
    a_src = attn_src[:, :D].reshape(1, HD).astype(f32)
    a_dst = attn_src[:, D:].reshape(1, HD).astype(f32)
    selT8 = jnp.kron(jnp.eye(H, dtype=f32), jnp.ones((1, D), f32))
    selT = jnp.concatenate([selT8, jnp.zeros((8, HD), f32)], axis=0)
    sel = selT.T

    fs, el16, er16 = pl.pallas_call(
        _proj_body,
        out_shape=[
            jax.ShapeDtypeStruct((N_SRC, HD), f32),
            jax.ShapeDtypeStruct((N_SRC, 16), f32),
            jax.ShapeDtypeStruct((N_DST, 16), f32),
        ],
    )(feat_src, feat_dst, W_src, b_src.reshape(1, HD), W_dst,
      b_dst.reshape(1, HD), a_src, a_dst, sel)

    src3 = edge_index[0].reshape(NW, NCH, C).astype(jnp.int32)
    dst3 = edge_index[1].reshape(NW, NCH, C).astype(jnp.int32)

    o_parts, d_parts = _edge_pass(fs, el16, er16, src3, dst3)

    out = pl.pallas_call(
        _final_body,
        out_shape=jax.ShapeDtypeStruct((N_DST, HD), f32),
    )(o_parts, d_parts, selT)
    return out


# speedup vs baseline: 58.7542x; 58.7542x over previous
import functools

import jax
import jax.numpy as jnp
from jax import lax
from jax.experimental import pallas as pl
from jax.experimental.pallas import tpu as pltpu
from jax.experimental.pallas import tpu_sc as plsc

N_SRC = 10000
N_DST = 10000
E = 320000
D_FEAT = 128
H = 8
D = 16
HD = H * D
NEG_SLOPE = 0.2

NC = 2
NS = 16
NW = NC * NS
EPW = E // NW
C = 80
NCH = EPW // C
STRIPE = 624
TAIL = N_DST - NS * STRIPE


def _proj_body(fsrc, fdst, Ws, bs, Wd, bd, asrc, adst, sel,
               fs_out, el_out, er_out):
    fs = jnp.dot(fsrc[...], Ws[...], preferred_element_type=jnp.float32)
    fs = fs + bs[...]
    fd = jnp.dot(fdst[...], Wd[...], preferred_element_type=jnp.float32)
    fd = fd + bd[...]
    fs_out[...] = fs
    el_out[...] = jnp.dot(fs * asrc[...], sel[...],
                          preferred_element_type=jnp.float32)
    er_out[...] = jnp.dot(fd * adst[...], sel[...],
                          preferred_element_type=jnp.float32)


_GATHER_DNUMS = lax.GatherDimensionNumbers(
    offset_dims=(), collapsed_slice_dims=(0,), start_index_map=(0,))


def _bcast_lane(v, h):
    idx = jnp.full((16, 1), h, dtype=jnp.int32)
    return lax.gather(v, idx, _GATHER_DNUMS, (1,),
                      mode=lax.GatherScatterMode.PROMISE_IN_BOUNDS)


def _edge_body(fs_hbm, el_hbm, er_hbm, src_hbm, dst_hbm,
               o_hbm, d_hbm,
               out_sh, den_sh, src_all, dst_all, elg, erg, rows, sem):
    c = lax.axis_index("c")
    s = lax.axis_index("s")
    wid = c * NS + s

    def _zrows(i, _):
        for h in range(8):
            rows[i, pl.ds(16 * h, 16)] = jnp.zeros((16,), jnp.float32)
        return 0
    lax.fori_loop(0, C, _zrows, 0)

    def _zelg(i, _):
        elg[i, :] = jnp.zeros((16,), jnp.float32)
        return 0
    lax.fori_loop(0, C, _zelg, 0)

    def _zo(t, _):
        pltpu.sync_copy(rows, out_sh.at[pl.ds(s * STRIPE + t * C, C)])
        return 0
    lax.fori_loop(0, 7, _zo, 0)
    pltpu.sync_copy(rows.at[pl.ds(0, STRIPE - 7 * C)],
                    out_sh.at[pl.ds(s * STRIPE + 7 * C, STRIPE - 7 * C)])

    def _zd(t, _):
        pltpu.sync_copy(elg, den_sh.at[pl.ds(s * STRIPE + t * C, C)])
        return 0
    lax.fori_loop(0, 7, _zd, 0)
    pltpu.sync_copy(elg.at[pl.ds(0, STRIPE - 7 * C)],
                    den_sh.at[pl.ds(s * STRIPE + 7 * C, STRIPE - 7 * C)])

    @pl.when(s == NS - 1)
    def _ztail():
        pltpu.sync_copy(rows.at[pl.ds(0, TAIL)],
                        out_sh.at[pl.ds(NS * STRIPE, TAIL)])
        pltpu.sync_copy(elg.at[pl.ds(0, TAIL)],
                        den_sh.at[pl.ds(NS * STRIPE, TAIL)])

    pltpu.sync_copy(src_hbm.at[wid], src_all)
    pltpu.sync_copy(dst_hbm.at[wid], dst_all)

    plsc.subcore_barrier()

    def _chunk(j, _):
        src_i = src_all.at[j]
        dst_i = dst_all.at[j]
        pltpu.async_copy(el_hbm.at[src_i], elg, sem).wait()
        pltpu.async_copy(er_hbm.at[dst_i], erg, sem).wait()
        pltpu.async_copy(fs_hbm.at[src_i], rows, sem).wait()

        def _wl(i, _):
            e = elg[i, :] + erg[i, :]
            e = jnp.maximum(e, 0.0) + NEG_SLOPE * jnp.minimum(e, 0.0)
            elg[i, :] = jnp.exp(e)
            return 0
        lax.fori_loop(0, C, _wl, 0)

        def _ml(i, _):
            wv = elg[i, :]
            for h in range(8):
                seg = rows[i, pl.ds(16 * h, 16)]
                rows[i, pl.ds(16 * h, 16)] = seg * _bcast_lane(wv, h)
            return 0
        lax.fori_loop(0, C, _ml, 0)

        pltpu.sync_copy(rows, out_sh.at[dst_i], add=True)
        pltpu.sync_copy(elg, den_sh.at[dst_i], add=True)
        return 0
    lax.fori_loop(0, NCH, _chunk, 0)

    plsc.subcore_barrier()

    pltpu.sync_copy(out_sh.at[pl.ds(s * STRIPE, STRIPE)],
                    o_hbm.at[c, pl.ds(s * STRIPE, STRIPE)])
    pltpu.sync_copy(den_sh.at[pl.ds(s * STRIPE, STRIPE)],
                    d_hbm.at[c, pl.ds(s * STRIPE, STRIPE)])

    @pl.when(s == NS - 1)
    def _wtail():
        pltpu.sync_copy(out_sh.at[pl.ds(NS * STRIPE, TAIL)],
                        o_hbm.at[c, pl.ds(NS * STRIPE, TAIL)])
        pltpu.sync_copy(den_sh.at[pl.ds(NS * STRIPE, TAIL)],
                        d_hbm.at[c, pl.ds(NS * STRIPE, TAIL)])


_edge_pass = functools.partial(
    pl.kernel,
    out_type=(
        jax.ShapeDtypeStruct((NC, N_DST, HD), jnp.float32),
        jax.ShapeDtypeStruct((NC, N_DST, 16), jnp.float32),
    ),
    mesh=plsc.VectorSubcoreMesh(core_axis_name="c", subcore_axis_name="s"),
    compiler_params=pltpu.CompilerParams(use_tc_tiling_on_sc=False),
    scratch_types=[
        pltpu.VMEM_SHARED((N_DST, HD), jnp.float32),
        pltpu.VMEM_SHARED((N_DST, 16), jnp.float32),
        pltpu.VMEM((NCH, C), jnp.int32),
        pltpu.VMEM((NCH, C), jnp.int32),
        pltpu.VMEM((C, 16), jnp.float32),
        pltpu.VMEM((C, 16), jnp.float32),
        pltpu.VMEM((C, HD), jnp.float32),
        pltpu.SemaphoreType.DMA,
    ],
)(_edge_body)


def _final_body(o_ref, d_ref, selT, out_ref):
    o = o_ref[0] + o_ref[1]
    den = d_ref[0] + d_ref[1]
    den128 = jnp.dot(den, selT[...], preferred_element_type=jnp.float32)
    out_ref[...] = o / (den128 + 1e-16)


def kernel(feat_src, feat_dst, edge_index, W_src, b_src, W_dst, b_dst,
           attn_src):
    f32 = jnp.float32
    a_src = attn_src[:, :D].reshape(1, HD).astype(f32)
    a_dst = attn_src[:, D:].reshape(1, HD).astype(f32)
    selT8 = jnp.kron(jnp.eye(H, dtype=f32), jnp.ones((1, D), f32))
    selT = jnp.concatenate([selT8, jnp.zeros((8, HD), f32)], axis=0)
    sel = selT.T

    fs, el16, er16 = pl.pallas_call(
        _proj_body,
        out_shape=[
            jax.ShapeDtypeStruct((N_SRC, HD), f32),
            jax.ShapeDtypeStruct((N_SRC, 16), f32),
            jax.ShapeDtypeStruct((N_DST, 16), f32),
        ],
    )(feat_src, feat_dst, W_src, b_src.reshape(1, HD), W_dst,
      b_dst.reshape(1, HD), a_src, a_dst, sel)

    src3 = edge_index[0].reshape(NW, NCH, C).astype(jnp.int32)
    dst3 = edge_index[1].reshape(NW, NCH, C).astype(jnp.int32)

    o_parts, d_parts = _edge_pass(fs, el16, er16, src3, dst3)

    out = pl.pallas_call(
        _final_body,
        out_shape=jax.ShapeDtypeStruct((N_DST, HD), f32),
    )(o_parts, d_parts, selT)
    return out
